# baseline (speedup 1.0000x reference)
import jax
import jax.numpy as jnp
from jax.experimental import pallas as pl
from jax.experimental.pallas import tpu as pltpu

D_MODEL = 768
NUM_EXPERTS = 64
BT = 4096
CH = 2048
NSTREAM = 2


def _top2_softmax(x_ref, w, bias, probs_ref, idx_ref):
    for c in range(BT // CH):
        logits = jax.lax.dot_general(
            w, x_ref[pl.ds(c * CH, CH), :],
            dimension_numbers=(((1,), (1,)), ((), ())),
            preferred_element_type=jnp.float32,
        ) + bias
        subl = jax.lax.broadcasted_iota(jnp.int32, logits.shape, 0)

        v0 = jnp.max(logits, axis=0, keepdims=True)
        i0 = jnp.min(jnp.where(logits == v0, subl, NUM_EXPERTS), axis=0,
                     keepdims=True)
        masked = jnp.where(subl == i0, -jnp.inf, logits)
        v1 = jnp.max(masked, axis=0, keepdims=True)
        i1 = jnp.min(jnp.where(masked == v1, subl, NUM_EXPERTS), axis=0,
                     keepdims=True)

        e = jnp.exp(v1 - v0)
        p0 = 1.0 / (1.0 + e)
        p1 = e * p0

        probs_ref[:, pl.ds(c * CH, CH)] = jnp.concatenate([p0, p1], axis=0)
        idx_ref[:, pl.ds(c * CH, CH)] = jnp.concatenate([i0, i1], axis=0)


def _router_kernel(*refs):
    x_refs = refs[:NSTREAM]
    w_ref, b_ref = refs[NSTREAM], refs[NSTREAM + 1]
    out_refs = refs[NSTREAM + 2:]
    w = w_ref[:]
    bias = b_ref[:]
    for s in range(NSTREAM):
        _top2_softmax(x_refs[s], w, bias, out_refs[2 * s], out_refs[2 * s + 1])


def kernel(x, W, b):
    n = x.shape[0]
    h = n // NSTREAM
    steps = h // BT
    in_specs = [
        pl.BlockSpec((BT, D_MODEL), lambda i, s=s: (i + s * steps, 0))
        for s in range(NSTREAM)
    ] + [
        pl.BlockSpec((NUM_EXPERTS, D_MODEL), lambda i: (0, 0)),
        pl.BlockSpec((NUM_EXPERTS, 1), lambda i: (0, 0)),
    ]
    out_specs = []
    out_shape = []
    for _ in range(NSTREAM):
        out_specs += [pl.BlockSpec((2, BT), lambda i: (0, i)),
                      pl.BlockSpec((2, BT), lambda i: (0, i))]
        out_shape += [jax.ShapeDtypeStruct((2, h), jnp.float32),
                      jax.ShapeDtypeStruct((2, h), jnp.int32)]
    outs = pl.pallas_call(
        _router_kernel,
        grid=(steps,),
        in_specs=in_specs,
        out_specs=out_specs,
        out_shape=out_shape,
        compiler_params=pltpu.CompilerParams(
            dimension_semantics=("arbitrary",),
        ),
    )(*([x] * NSTREAM), W, b.reshape(NUM_EXPERTS, 1))
    probs_t = jnp.concatenate(outs[0::2], axis=1)
    idx_t = jnp.concatenate(outs[1::2], axis=1)
    return (probs_t.T, idx_t.T)


# speedup vs baseline: 1.1824x; 1.1824x over previous
import jax
import jax.numpy as jnp
from jax.experimental import pallas as pl
from jax.experimental.pallas import tpu as pltpu

D_MODEL = 768
NUM_EXPERTS = 64
BT = 4096
CH = 2048
NSTREAM = 1


def _top2_softmax(x_ref, w, bias, probs_ref, idx_ref):
    for c in range(BT // CH):
        logits = jax.lax.dot_general(
            w, x_ref[pl.ds(c * CH, CH), :],
            dimension_numbers=(((1,), (1,)), ((), ())),
            preferred_element_type=jnp.float32,
        ) + bias
        subl = jax.lax.broadcasted_iota(jnp.int32, logits.shape, 0)

        v0 = jnp.max(logits, axis=0, keepdims=True)
        i0 = jnp.min(jnp.where(logits == v0, subl, NUM_EXPERTS), axis=0,
                     keepdims=True)
        masked = jnp.where(subl == i0, -jnp.inf, logits)
        v1 = jnp.max(masked, axis=0, keepdims=True)
        i1 = jnp.min(jnp.where(masked == v1, subl, NUM_EXPERTS), axis=0,
                     keepdims=True)

        e = jnp.exp(v1 - v0)
        p0 = 1.0 / (1.0 + e)
        p1 = e * p0

        probs_ref[:, pl.ds(c * CH, CH)] = jnp.concatenate([p0, p1], axis=0)
        idx_ref[:, pl.ds(c * CH, CH)] = jnp.concatenate([i0, i1], axis=0)


def _router_kernel(*refs):
    x_refs = refs[:NSTREAM]
    w_ref, b_ref = refs[NSTREAM], refs[NSTREAM + 1]
    out_refs = refs[NSTREAM + 2:]
    w = w_ref[:]
    bias = b_ref[:]
    for s in range(NSTREAM):
        _top2_softmax(x_refs[s], w, bias, out_refs[2 * s], out_refs[2 * s + 1])


def kernel(x, W, b):
    n = x.shape[0]
    h = n // NSTREAM
    steps = h // BT
    in_specs = [
        pl.BlockSpec((BT, D_MODEL), lambda i, s=s: (i + s * steps, 0))
        for s in range(NSTREAM)
    ] + [
        pl.BlockSpec((NUM_EXPERTS, D_MODEL), lambda i: (0, 0)),
        pl.BlockSpec((NUM_EXPERTS, 1), lambda i: (0, 0)),
    ]
    out_specs = []
    out_shape = []
    for _ in range(NSTREAM):
        out_specs += [pl.BlockSpec((2, BT), lambda i: (0, i)),
                      pl.BlockSpec((2, BT), lambda i: (0, i))]
        out_shape += [jax.ShapeDtypeStruct((2, h), jnp.float32),
                      jax.ShapeDtypeStruct((2, h), jnp.int32)]
    outs = pl.pallas_call(
        _router_kernel,
        grid=(steps,),
        in_specs=in_specs,
        out_specs=out_specs,
        out_shape=out_shape,
        compiler_params=pltpu.CompilerParams(
            dimension_semantics=("arbitrary",),
        ),
    )(*([x] * NSTREAM), W, b.reshape(NUM_EXPERTS, 1))
    probs_t = jnp.concatenate(outs[0::2], axis=1)
    idx_t = jnp.concatenate(outs[1::2], axis=1)
    return (probs_t.T, idx_t.T)
